# baseline (speedup 1.0000x reference)
import jax
import jax.numpy as jnp
from jax.experimental import pallas as pl


def _add_kernel(x_ref, p_ref, o_ref):
    o_ref[...] = x_ref[...] + p_ref[...][None, :, :]


def kernel(inputs, position_embeddings):
    B, S, D = inputs.shape
    S_BLK = 512
    grid = (S // S_BLK,)
    return pl.pallas_call(
        _add_kernel,
        grid=grid,
        in_specs=[
            pl.BlockSpec((B, S_BLK, D), lambda i: (0, i, 0)),
            pl.BlockSpec((S_BLK, D), lambda i: (i, 0)),
        ],
        out_specs=pl.BlockSpec((B, S_BLK, D), lambda i: (0, i, 0)),
        out_shape=jax.ShapeDtypeStruct((B, S, D), inputs.dtype),
    )(inputs, position_embeddings)
